# baseline (speedup 1.0000x reference)
import jax
import jax.numpy as jnp
from jax import lax
from jax.experimental import pallas as pl
from jax.experimental.pallas import tpu as pltpu
from jax.experimental.pallas import tpu_sc as plsc

_MAXLEN = 200
_EMBED = 64
_BATCH = 4096
_LANES = 16

_NC = 2
_NS = 16
_NW = _NC * _NS
_BPW = _BATCH // _NW
_G = 4
_ROWS = _G * _MAXLEN
_GATHER = 80
_NG = _ROWS // _GATHER
_CHUNKS = _BPW // _G
_J = _EMBED // _LANES


def _tec_body(x_hbm, tok_hbm, pos_hbm, out_hbm, pos_v, idx_v, rows_v,
              gsem0, gsem1, osem0, osem1):
    c = lax.axis_index("c")
    s = lax.axis_index("s")
    wid = s * _NC + c
    gsems = (gsem0, gsem1)
    osems = (osem0, osem1)
    pltpu.sync_copy(pos_hbm, pos_v)
    elem_base = wid * _BPW

    def add_positions(slot):
        @plsc.parallel_loop(0, _MAXLEN, unroll=2)
        def _(t):
            for j in range(_J):
                p = pos_v[t, pl.ds(j * _LANES, _LANES)]
                for b in range(_G):
                    r = b * _MAXLEN + t
                    rows_v[slot, r, pl.ds(j * _LANES, _LANES)] = (
                        rows_v[slot, r, pl.ds(j * _LANES, _LANES)] + p
                    )

    gds = [None] * _CHUNKS
    ods = [None] * _CHUNKS
    for i in range(_CHUNKS + 1):
        if i < _CHUNKS:
            slot = i % 2
            if i >= 2:
                for d in ods[i - 2]:
                    d.wait()
            b0 = elem_base + i * _G
            for g in range(_G):
                pltpu.sync_copy(x_hbm.at[b0 + g],
                                idx_v.at[slot, pl.ds(g * _MAXLEN, _MAXLEN)])
            gds[i] = [
                pltpu.async_copy(
                    tok_hbm.at[idx_v.at[slot, pl.ds(k * _GATHER, _GATHER)]],
                    rows_v.at[slot, pl.ds(k * _GATHER, _GATHER)],
                    gsems[slot],
                )
                for k in range(_NG)
            ]
        if i >= 1:
            k = i - 1
            slot = k % 2
            for d in gds[k]:
                d.wait()
            add_positions(slot)
            kb0 = elem_base + k * _G
            ods[k] = [
                pltpu.async_copy(
                    rows_v.at[slot, pl.ds(g * _MAXLEN, _MAXLEN)],
                    out_hbm.at[kb0 + g],
                    osems[slot],
                )
                for g in range(_G)
            ]
    for d in ods[_CHUNKS - 2]:
        d.wait()
    for d in ods[_CHUNKS - 1]:
        d.wait()


def kernel(x, token_table, pos_table):
    x2d = x.astype(jnp.int32)
    mesh = plsc.VectorSubcoreMesh(core_axis_name="c", subcore_axis_name="s")
    out = pl.kernel(
        _tec_body,
        out_type=jax.ShapeDtypeStruct((_BATCH, _MAXLEN, _EMBED), jnp.float32),
        mesh=mesh,
        compiler_params=pltpu.CompilerParams(use_tc_tiling_on_sc=False),
        scratch_types=[
            pltpu.VMEM((_MAXLEN, _EMBED), jnp.float32),
            pltpu.VMEM((2, _ROWS), jnp.int32),
            pltpu.VMEM((2, _ROWS, _EMBED), jnp.float32),
            pltpu.SemaphoreType.DMA,
            pltpu.SemaphoreType.DMA,
            pltpu.SemaphoreType.DMA,
            pltpu.SemaphoreType.DMA,
        ],
    )(x2d, token_table, pos_table)
    return out


# speedup vs baseline: 1.1970x; 1.1970x over previous
import jax
import jax.numpy as jnp
from jax import lax
from jax.experimental import pallas as pl
from jax.experimental.pallas import tpu as pltpu
from jax.experimental.pallas import tpu_sc as plsc

_MAXLEN = 200
_EMBED = 64
_VOCAB = 100000
_BATCH = 4096
_LANES = 16
_PADDED = 128

_NC = 2
_NS = 16
_NW = _NC * _NS
_BPW = _BATCH // _NW
_ROUNDS = _BPW // 2
_J = _EMBED // _LANES


def _tec_body(x_hbm, tok_hbm, pos_hbm, out_hbm, pos_v, idx_v, rows_v, outs_v,
              g0, g1, o0, o1):
    c = lax.axis_index("c")
    s = lax.axis_index("s")
    wid = s * _NC + c
    gsems = (g0, g1)
    osems = (o0, o1)
    pltpu.sync_copy(pos_hbm, pos_v)
    elem_base = wid * _BPW

    def fire(slot, b):
        i0 = slot * _MAXLEN
        pltpu.sync_copy(x_hbm.at[pl.ds(b * _MAXLEN, _MAXLEN)],
                        idx_v.at[pl.ds(i0, _MAXLEN)])
        pltpu.async_copy(tok_hbm.at[idx_v.at[pl.ds(i0, 128)]],
                         rows_v.at[slot, pl.ds(0, 128)], gsems[slot])
        pltpu.async_copy(tok_hbm.at[idx_v.at[pl.ds(i0 + 128, 72)]],
                         rows_v.at[slot, pl.ds(128, 72)], gsems[slot])

    def wait_gathers(slot):
        pltpu.make_async_copy(tok_hbm.at[pl.ds(0, _MAXLEN)],
                              rows_v.at[slot], gsems[slot]).wait()

    def drain_out(slot, b):
        pltpu.make_async_copy(outs_v.at[slot], out_hbm.at[b],
                              osems[slot]).wait()

    def add_positions(slot):
        @plsc.parallel_loop(0, _MAXLEN, unroll=2)
        def _(t):
            for jj in range(_J):
                p = pos_v[pl.ds(t * _EMBED + jj * _LANES, _LANES)]
                outs_v[slot, t, pl.ds(jj * _LANES, _LANES)] = (
                    rows_v[slot, t, pl.ds(jj * _LANES, _LANES)] + p
                )

    def proc(slot, b, drain_pred):
        wait_gathers(slot)

        @pl.when(drain_pred)
        def _():
            drain_out(slot, b - 2)

        add_positions(slot)
        pltpu.async_copy(outs_v.at[slot], out_hbm.at[b], osems[slot])

    def round_body(r, carry):
        b0 = elem_base + r * 2
        fire(0, b0)

        @pl.when(r >= 1)
        def _():
            proc(1, b0 - 1, r >= 2)

        fire(1, b0 + 1)
        proc(0, b0, r >= 1)
        return carry

    lax.fori_loop(0, _ROUNDS, round_body, 0)
    last = elem_base + _BPW - 1
    proc(1, last, True)
    drain_out(0, last - 1)
    drain_out(1, last)


def kernel(x, token_table, pos_table):
    x1d = x.reshape(-1).astype(jnp.int32)
    tok128 = jnp.pad(token_table, ((0, 0), (0, _PADDED - _EMBED)))
    pos1d = pos_table.reshape(-1)
    mesh = plsc.VectorSubcoreMesh(core_axis_name="c", subcore_axis_name="s")
    out = pl.kernel(
        _tec_body,
        out_type=jax.ShapeDtypeStruct((_BATCH, _MAXLEN, _EMBED), jnp.float32),
        mesh=mesh,
        compiler_params=pltpu.CompilerParams(use_tc_tiling_on_sc=True),
        scratch_types=[
            pltpu.VMEM((_MAXLEN * _EMBED,), jnp.float32),
            pltpu.VMEM((2 * _MAXLEN,), jnp.int32),
            pltpu.VMEM((2, _MAXLEN, _PADDED), jnp.float32),
            pltpu.VMEM((2, _MAXLEN, _EMBED), jnp.float32),
            pltpu.SemaphoreType.DMA,
            pltpu.SemaphoreType.DMA,
            pltpu.SemaphoreType.DMA,
            pltpu.SemaphoreType.DMA,
        ],
    )(x1d, tok128, pos1d)
    return out
